# baseline (speedup 1.0000x reference)
import functools
import math

import jax
import jax.numpy as jnp
from jax import lax
from jax.experimental import pallas as pl
from jax.experimental.pallas import tpu as pltpu
from jax.experimental.pallas import tpu_sc as plsc

N = 5000
C = 91
NCLS = C - 1
K = 100
NPAD = 5120
KPAD = 112
KOUT = 104
LANE_BLK = 1024
IMG_H, IMG_W = 800.0, 1333.0
SCORE_THRESH = 0.05
NMS_THRESH = 0.5
MIN_SIZE = 1.0
BIG_NEG = -1e9
HALF_BIG_NEG = -5e8
W_XY = 10.0
W_WH = 5.0
BBOX_XFORM_CLIP = float(math.log(1000.0 / 16.0))

_NC = 2
_NS = 16
L = 16
NW = _NC * _NS
NBLK = NPAD // L
CPW = -(-NCLS // NW)


def _softmax_body(logit_ref, probT_ref):
    x = logit_ref[...]
    m = jnp.max(x, axis=-1, keepdims=True)
    e = jnp.exp(x - m)
    p = e / jnp.sum(e, axis=-1, keepdims=True)
    probT_ref[:, :N] = p[:, 1:].T


def _softmax_call(class_logit):
    return pl.pallas_call(
        _softmax_body,
        out_shape=jax.ShapeDtypeStruct((NCLS, NPAD), jnp.float32),
    )(class_logit)


def _decode_plane(dx_r, dy_r, dw_r, dh_r, w, h, cx, cy):
    dx = dx_r[...] / W_XY
    dy = dy_r[...] / W_XY
    dw = jnp.minimum(dw_r[...] / W_WH, BBOX_XFORM_CLIP)
    dh = jnp.minimum(dh_r[...] / W_WH, BBOX_XFORM_CLIP)
    pcx = dx * w + cx
    pcy = dy * h + cy
    pw = jnp.exp(dw) * w
    ph = jnp.exp(dh) * h
    x1 = jnp.clip(pcx - 0.5 * pw, 0.0, IMG_W)
    y1 = jnp.clip(pcy - 0.5 * ph, 0.0, IMG_H)
    x2 = jnp.clip(pcx + 0.5 * pw, 0.0, IMG_W)
    y2 = jnp.clip(pcy + 0.5 * ph, 0.0, IMG_H)
    return x1, y1, x2, y2


def _prep_body(score_ref, bdx, bdy, bdw, bdh, tdx, tdy, tdw, tdh, prop_ref,
               s_ref, x1_ref, y1_ref, x2_ref, y2_ref,
               fx1_ref, fy1_ref, fx2_ref, fy2_ref):
    i = pl.program_id(0)
    gcol = i * LANE_BLK + lax.broadcasted_iota(jnp.int32, (1, LANE_BLK), 1)
    valid_col = gcol < N
    px1 = prop_ref[0:1, :]
    py1 = prop_ref[1:2, :]
    px2 = prop_ref[2:3, :]
    py2 = prop_ref[3:4, :]
    w = px2 - px1
    h = py2 - py1
    cx = px1 + 0.5 * w
    cy = py1 + 0.5 * h

    x1, y1, x2, y2 = _decode_plane(bdx, bdy, bdw, bdh, w, h, cx, cy)
    f1, g1, f2, g2 = _decode_plane(tdx, tdy, tdw, tdh, w, h, cx, cy)

    sc = score_ref[...]
    bw = x2 - x1
    bh = y2 - y1
    keep = (sc >= SCORE_THRESH) & (bw >= MIN_SIZE) & (bh >= MIN_SIZE) & valid_col
    s_ref[...] = jnp.where(keep, sc, BIG_NEG)
    x1_ref[...] = x1
    y1_ref[...] = y1
    x2_ref[...] = x2
    y2_ref[...] = y2
    fx1_ref[...] = f1
    fy1_ref[...] = g1
    fx2_ref[...] = f2
    fy2_ref[...] = g2


def _prep_call(scoresT, deltas, tdeltas, propT):
    nblk = NPAD // LANE_BLK
    cls_spec = pl.BlockSpec((NCLS, LANE_BLK), lambda i: (0, i))
    prop_spec = pl.BlockSpec((4, LANE_BLK), lambda i: (0, i))
    return pl.pallas_call(
        _prep_body,
        grid=(nblk,),
        in_specs=[cls_spec] * 9 + [prop_spec],
        out_specs=[cls_spec] * 9,
        out_shape=[jax.ShapeDtypeStruct((NCLS, NPAD), jnp.float32)] * 9,
    )(scoresT, *deltas, *tdeltas, propT)


def _nms_body(s_hbm, x1_hbm, y1_hbm, x2_hbm, y2_hbm,
              fx1_hbm, fy1_hbm, fx2_hbm, fy2_hbm,
              os_hbm, obox_hbm, ofut_hbm,
              stg_s, stg_x1, stg_y1, stg_x2, stg_y2,
              stg_f1, stg_f2, stg_f3, stg_f4,
              c_s, c_x1, c_y1, c_x2, c_y2, c_area, c_oidxf,
              o_s, o_ki, o_box, o_fut,
              sem, sem_in, sem_fut):
    wid = lax.axis_index("s") * _NC + lax.axis_index("c")
    lanes = lax.iota(jnp.int32, L)
    lane0 = lanes == 0
    zeros = jnp.zeros((L,), jnp.float32)
    negs = jnp.full((L,), BIG_NEG, jnp.float32)
    ninf = jnp.full((L,), -3e38, jnp.float32)
    bigi = jnp.full((L,), 2**31 - 1, jnp.int32)

    def put1(ref, idx, val):
        plsc.store_scatter(ref, [jnp.full((L,), idx, jnp.int32)],
                           jnp.full((L,), val, jnp.float32), mask=lane0)

    def _xperm(vv, perm):
        return lax.gather(
            vv, perm[:, None],
            lax.GatherDimensionNumbers((), (0,), (0,)),
            (1,), mode=lax.GatherScatterMode.PROMISE_IN_BOUNDS)

    descs_in = [
        [pltpu.make_async_copy(h.at[wid + t * NW], d, sem_in)
         for h, d in ((s_hbm, stg_s), (x1_hbm, stg_x1), (y1_hbm, stg_y1),
                      (x2_hbm, stg_x2), (y2_hbm, stg_y2))]
        for t in range(CPW)
    ]
    descs_fut = [
        [pltpu.make_async_copy(h.at[wid + t * NW], d, sem_fut)
         for h, d in ((fx1_hbm, stg_f1), (fy1_hbm, stg_f2),
                      (fx2_hbm, stg_f3), (fy2_hbm, stg_f4))]
        for t in range(CPW)
    ]

    def do_class(cls, t):
        for cp in descs_in[t]:
            cp.wait()

        def comp_blk(b, st):
            cnt, vmax, vidx = st
            off = b * L
            sv = stg_s[pl.ds(off, L)]
            msk = sv > HALF_BIG_NEG
            mi = msk.astype(jnp.int32)
            ns = plsc.all_reduce_population_count(msk)[0]

            def scat(args):
                cnt, vmax, vidx = args
                pos = (cnt - 1) + plsc.cumsum(mi)
                x1v = stg_x1[pl.ds(off, L)]
                y1v = stg_y1[pl.ds(off, L)]
                x2v = stg_x2[pl.ds(off, L)]
                y2v = stg_y2[pl.ds(off, L)]
                plsc.store_scatter(c_s, [pos], sv, mask=msk)
                plsc.store_scatter(c_x1, [pos], x1v, mask=msk)
                plsc.store_scatter(c_y1, [pos], y1v, mask=msk)
                plsc.store_scatter(c_x2, [pos], x2v, mask=msk)
                plsc.store_scatter(c_y2, [pos], y2v, mask=msk)
                plsc.store_scatter(c_area, [pos], (x2v - x1v) * (y2v - y1v), mask=msk)
                plsc.store_scatter(c_oidxf, [pos], (off + lanes).astype(jnp.float32), mask=msk)
                upd = msk & (sv > vmax)
                return (jnp.where(upd, sv, vmax), jnp.where(upd, pos, vidx))

            vmax, vidx = lax.cond(ns > 0, scat, lambda a: (a[1], a[2]),
                                  (cnt, vmax, vidx))
            return cnt + ns, vmax, vidx

        def comp_blk2(b, st):
            st = comp_blk(2 * b, st)
            return comp_blk(2 * b + 1, st)

        cnt, vmax, vidx = lax.fori_loop(0, NBLK // 2, comp_blk2,
                                        (jnp.int32(0), ninf, bigi))
        c_s[pl.ds(cnt, L)] = negs
        c_s[pl.ds(cnt + L, L)] = negs
        c_s[pl.ds(cnt + 2 * L, L)] = negs
        c_s[pl.ds(cnt + 3 * L, L)] = negs
        nb2 = (cnt + (4 * L - 1)) // (4 * L)

        if t + 1 < CPW:
            @pl.when(cls + NW < NCLS)
            def _():
                for cp in descs_in[t + 1]:
                    cp.start()

        for j in range(KPAD // L):
            o_s[pl.ds(j * L, L)] = zeros

        bs0 = jnp.max(vmax)
        bi0 = jnp.min(jnp.where(vmax == bs0, vidx, 2**31 - 1))

        def cond(st):
            k, bi, bs = st
            return (k < K) & (bs > HALF_BIG_NEG)

        def body(st):
            k, bi, bs = st
            biv = jnp.full((L,), bi, jnp.int32)
            x1cv = plsc.load_gather(c_x1, [biv])
            y1cv = plsc.load_gather(c_y1, [biv])
            x2cv = plsc.load_gather(c_x2, [biv])
            y2cv = plsc.load_gather(c_y2, [biv])
            acv = plsc.load_gather(c_area, [biv])
            put1(o_s, k, bs)
            put1(o_ki, k, bi.astype(jnp.float32))
            put1(c_s, bi, BIG_NEG)

            def sup_half(off, vmax, vidx):
                sv = c_s[pl.ds(off, L)]
                x1v = c_x1[pl.ds(off, L)]
                y1v = c_y1[pl.ds(off, L)]
                x2v = c_x2[pl.ds(off, L)]
                y2v = c_y2[pl.ds(off, L)]
                av = c_area[pl.ds(off, L)]
                iw = jnp.maximum(jnp.minimum(x2cv, x2v) - jnp.maximum(x1cv, x1v), 0.0)
                ih = jnp.maximum(jnp.minimum(y2cv, y2v) - jnp.maximum(y1cv, y1v), 0.0)
                inter = iw * ih
                sup = inter > NMS_THRESH * (acv + av - inter + 1e-9)
                sv = jnp.where(sup, BIG_NEG, sv)
                c_s[pl.ds(off, L)] = sv
                upd = sv > vmax
                return jnp.where(upd, sv, vmax), jnp.where(upd, off + lanes, vidx)

            def sup_blk(b, carry):
                vmax, vidx = carry
                off = b * (4 * L)
                vmax, vidx = sup_half(off, vmax, vidx)
                vmax, vidx = sup_half(off + L, vmax, vidx)
                vmax, vidx = sup_half(off + 2 * L, vmax, vidx)
                return sup_half(off + 3 * L, vmax, vidx)

            vmax, vidx = lax.fori_loop(0, nb2, sup_blk, (ninf, bigi))
            v, ii = vmax, vidx
            for d in (1, 2, 4, 8):
                pm = lanes ^ d
                pv = _xperm(v, pm)
                pi = _xperm(ii, pm)
                better = (pv > v) | ((pv == v) & (pi < ii))
                v = jnp.where(better, pv, v)
                ii = jnp.where(better, pi, ii)
            bs2 = v[0]
            bi2 = ii.astype(jnp.float32)[0].astype(jnp.int32)
            return k + 1, bi2, bs2

        kfin, _, _ = lax.while_loop(cond, body, (jnp.int32(0), bi0, bs0))

        for cp in descs_fut[t]:
            cp.wait()

        for j in range(KPAD // L):
            kv = j * L + lanes
            vm = kv < kfin
            kiv = o_ki[pl.ds(j * L, L)]
            biv = jnp.where(vm, kiv.astype(jnp.int32), 0)
            gof = plsc.load_gather(c_oidxf, [biv], mask=vm)
            oiv = jnp.where(vm, gof.astype(jnp.int32), 0)
            kv4 = kv * 4
            for comp, src_ref in enumerate((c_x1, c_y1, c_x2, c_y2)):
                g = plsc.load_gather(src_ref, [biv], mask=vm)
                plsc.store_scatter(o_box, [kv4 + comp], jnp.where(vm, g, 0.0))
            for comp, src_ref in enumerate((stg_f1, stg_f2, stg_f3, stg_f4)):
                g = plsc.load_gather(src_ref, [oiv], mask=vm)
                plsc.store_scatter(o_fut, [kv4 + comp], jnp.where(vm, g, 0.0))

        ocps = [
            pltpu.async_copy(o_s, os_hbm.at[cls], sem),
            pltpu.async_copy(o_box, obox_hbm.at[cls], sem),
            pltpu.async_copy(o_fut, ofut_hbm.at[cls], sem),
        ]
        if t + 1 < CPW:
            @pl.when(cls + NW < NCLS)
            def _():
                for cp in descs_fut[t + 1]:
                    cp.start()

        for cp in ocps:
            cp.wait()

    for cp in descs_in[0]:
        cp.start()
    for cp in descs_fut[0]:
        cp.start()
    for t in range(CPW):
        cls = wid + t * NW

        @pl.when(cls < NCLS)
        def _(t=t, cls=cls):
            do_class(cls, t)


def _nms_call(planes):
    mesh = plsc.VectorSubcoreMesh(core_axis_name="c", subcore_axis_name="s", num_cores=2, num_subcores=16)
    f32 = jnp.float32
    out_type = [jax.ShapeDtypeStruct((NCLS, KPAD), f32),
                jax.ShapeDtypeStruct((NCLS, KPAD * 4), f32),
                jax.ShapeDtypeStruct((NCLS, KPAD * 4), f32)]
    scratch = (
        [pltpu.VMEM((NPAD,), f32)] * 9
        + [pltpu.VMEM((NPAD + 4 * L,), f32)] * 7
        + [pltpu.VMEM((KPAD,), f32)] * 2
        + [pltpu.VMEM((KPAD * 4,), f32)] * 2
        + [pltpu.SemaphoreType.DMA] * 3
    )
    fn = pl.kernel(
        _nms_body,
        out_type=out_type,
        mesh=mesh,
        scratch_types=scratch,
        compiler_params=pltpu.CompilerParams(needs_layout_passes=False),
    )
    return fn(*planes)


def kernel(class_logit, box_regression, trajectory_regression, proposal):
    scoresT = _softmax_call(class_logit)

    pad = NPAD - N

    d3 = jnp.stack([box_regression.reshape(N, C, 4),
                    trajectory_regression.reshape(N, C, 4)])[:, :, 1:, :]
    d3 = jnp.pad(jnp.transpose(d3, (0, 3, 2, 1)),
                 ((0, 0), (0, 0), (0, 0), (0, pad)))
    deltas = (d3[0, 0], d3[0, 1], d3[0, 2], d3[0, 3])
    tdeltas = (d3[1, 0], d3[1, 1], d3[1, 2], d3[1, 3])
    propT = jnp.pad(proposal.T, ((0, 0), (0, pad)))

    planes = _prep_call(scoresT, deltas, tdeltas, propT)
    os_, boxf, futf = _nms_call(planes)

    boxes = boxf[:, :K * 4].reshape(NCLS, K, 4)
    fut = futf[:, :K * 4].reshape(NCLS, K, 4)
    scores = os_[:, :K]
    labels = jnp.broadcast_to(
        jnp.arange(1, C, dtype=jnp.int32)[:, None], (NCLS, K))
    return boxes, scores, labels, fut


# speedup vs baseline: 1.0233x; 1.0233x over previous
import functools
import math

import jax
import jax.numpy as jnp
from jax import lax
from jax.experimental import pallas as pl
from jax.experimental.pallas import tpu as pltpu
from jax.experimental.pallas import tpu_sc as plsc

N = 5000
C = 91
NCLS = C - 1
K = 100
NPAD = 5120
KPAD = 112
KOUT = 104
LANE_BLK = 1024
IMG_H, IMG_W = 800.0, 1333.0
SCORE_THRESH = 0.05
NMS_THRESH = 0.5
MIN_SIZE = 1.0
BIG_NEG = -1e9
HALF_BIG_NEG = -5e8
W_XY = 10.0
W_WH = 5.0
BBOX_XFORM_CLIP = float(math.log(1000.0 / 16.0))

_NC = 2
_NS = 16
L = 16
NW = _NC * _NS
NBLK = NPAD // L
CPW = -(-NCLS // NW)


def _softmax_body(logit_ref, probT_ref):
    x = logit_ref[...]
    m = jnp.max(x, axis=-1, keepdims=True)
    e = jnp.exp(x - m)
    p = e / jnp.sum(e, axis=-1, keepdims=True)
    probT_ref[:, :N] = p[:, 1:].T


def _softmax_call(class_logit):
    return pl.pallas_call(
        _softmax_body,
        out_shape=jax.ShapeDtypeStruct((NCLS, NPAD), jnp.float32),
    )(class_logit)


def _decode_plane(dx_r, dy_r, dw_r, dh_r, w, h, cx, cy):
    dx = dx_r[...] / W_XY
    dy = dy_r[...] / W_XY
    dw = jnp.minimum(dw_r[...] / W_WH, BBOX_XFORM_CLIP)
    dh = jnp.minimum(dh_r[...] / W_WH, BBOX_XFORM_CLIP)
    pcx = dx * w + cx
    pcy = dy * h + cy
    pw = jnp.exp(dw) * w
    ph = jnp.exp(dh) * h
    x1 = jnp.clip(pcx - 0.5 * pw, 0.0, IMG_W)
    y1 = jnp.clip(pcy - 0.5 * ph, 0.0, IMG_H)
    x2 = jnp.clip(pcx + 0.5 * pw, 0.0, IMG_W)
    y2 = jnp.clip(pcy + 0.5 * ph, 0.0, IMG_H)
    return x1, y1, x2, y2


def _prep_body(score_ref, bdx, bdy, bdw, bdh, tdx, tdy, tdw, tdh, prop_ref,
               s_ref, x1_ref, y1_ref, x2_ref, y2_ref,
               fx1_ref, fy1_ref, fx2_ref, fy2_ref):
    i = pl.program_id(0)
    gcol = i * LANE_BLK + lax.broadcasted_iota(jnp.int32, (1, LANE_BLK), 1)
    valid_col = gcol < N
    px1 = prop_ref[0:1, :]
    py1 = prop_ref[1:2, :]
    px2 = prop_ref[2:3, :]
    py2 = prop_ref[3:4, :]
    w = px2 - px1
    h = py2 - py1
    cx = px1 + 0.5 * w
    cy = py1 + 0.5 * h

    x1, y1, x2, y2 = _decode_plane(bdx, bdy, bdw, bdh, w, h, cx, cy)
    f1, g1, f2, g2 = _decode_plane(tdx, tdy, tdw, tdh, w, h, cx, cy)

    sc = score_ref[...]
    bw = x2 - x1
    bh = y2 - y1
    keep = (sc >= SCORE_THRESH) & (bw >= MIN_SIZE) & (bh >= MIN_SIZE) & valid_col
    s_ref[...] = jnp.where(keep, sc, BIG_NEG)
    x1_ref[...] = x1
    y1_ref[...] = y1
    x2_ref[...] = x2
    y2_ref[...] = y2
    fx1_ref[...] = f1
    fy1_ref[...] = g1
    fx2_ref[...] = f2
    fy2_ref[...] = g2


def _prep_call(scoresT, deltas, tdeltas, propT):
    nblk = NPAD // LANE_BLK
    cls_spec = pl.BlockSpec((NCLS, LANE_BLK), lambda i: (0, i))
    prop_spec = pl.BlockSpec((4, LANE_BLK), lambda i: (0, i))
    return pl.pallas_call(
        _prep_body,
        grid=(nblk,),
        in_specs=[cls_spec] * 9 + [prop_spec],
        out_specs=[cls_spec] * 9,
        out_shape=[jax.ShapeDtypeStruct((NCLS, NPAD), jnp.float32)] * 9,
    )(scoresT, *deltas, *tdeltas, propT)


def _nms_body(s_hbm, x1_hbm, y1_hbm, x2_hbm, y2_hbm,
              fx1_hbm, fy1_hbm, fx2_hbm, fy2_hbm,
              os_hbm, obox_hbm, ofut_hbm,
              stg_s, stg_x1, stg_y1, stg_x2, stg_y2,
              stg_f1, stg_f2, stg_f3, stg_f4,
              c_s, c_x1, c_y1, c_x2, c_y2, c_area, c_oidxf,
              o_s, o_ki, o_box, o_fut,
              sem, sem_in, sem_fut):
    wid = lax.axis_index("s") * _NC + lax.axis_index("c")
    lanes = lax.iota(jnp.int32, L)
    lane0 = lanes == 0
    zeros = jnp.zeros((L,), jnp.float32)
    negs = jnp.full((L,), BIG_NEG, jnp.float32)
    ninf = jnp.full((L,), -3e38, jnp.float32)
    bigi = jnp.full((L,), 2**31 - 1, jnp.int32)

    def put1(ref, idx, val):
        plsc.store_scatter(ref, [jnp.full((L,), idx, jnp.int32)],
                           jnp.full((L,), val, jnp.float32), mask=lane0)

    descs_in = [
        [pltpu.make_async_copy(h.at[wid + t * NW], d, sem_in)
         for h, d in ((s_hbm, stg_s), (x1_hbm, stg_x1), (y1_hbm, stg_y1),
                      (x2_hbm, stg_x2), (y2_hbm, stg_y2))]
        for t in range(CPW)
    ]
    descs_fut = [
        [pltpu.make_async_copy(h.at[wid + t * NW], d, sem_fut)
         for h, d in ((fx1_hbm, stg_f1), (fy1_hbm, stg_f2),
                      (fx2_hbm, stg_f3), (fy2_hbm, stg_f4))]
        for t in range(CPW)
    ]

    def do_class(cls, t):
        for cp in descs_in[t]:
            cp.wait()

        def comp_blk(b, st):
            cnt, vmax, vidx = st
            off = b * L
            sv = stg_s[pl.ds(off, L)]
            msk = sv > HALF_BIG_NEG
            mi = msk.astype(jnp.int32)
            ns = plsc.all_reduce_population_count(msk)[0]

            def scat(args):
                cnt, vmax, vidx = args
                pos = (cnt - 1) + plsc.cumsum(mi)
                x1v = stg_x1[pl.ds(off, L)]
                y1v = stg_y1[pl.ds(off, L)]
                x2v = stg_x2[pl.ds(off, L)]
                y2v = stg_y2[pl.ds(off, L)]
                plsc.store_scatter(c_s, [pos], sv, mask=msk)
                plsc.store_scatter(c_x1, [pos], x1v, mask=msk)
                plsc.store_scatter(c_y1, [pos], y1v, mask=msk)
                plsc.store_scatter(c_x2, [pos], x2v, mask=msk)
                plsc.store_scatter(c_y2, [pos], y2v, mask=msk)
                plsc.store_scatter(c_area, [pos], (x2v - x1v) * (y2v - y1v), mask=msk)
                plsc.store_scatter(c_oidxf, [pos], (off + lanes).astype(jnp.float32), mask=msk)
                upd = msk & (sv > vmax)
                return (jnp.where(upd, sv, vmax), jnp.where(upd, pos, vidx))

            vmax, vidx = lax.cond(ns > 0, scat, lambda a: (a[1], a[2]),
                                  (cnt, vmax, vidx))
            return cnt + ns, vmax, vidx

        def comp_blk2(b, st):
            st = comp_blk(2 * b, st)
            return comp_blk(2 * b + 1, st)

        cnt, vmax, vidx = lax.fori_loop(0, NBLK // 2, comp_blk2,
                                        (jnp.int32(0), ninf, bigi))
        c_s[pl.ds(cnt, L)] = negs
        c_s[pl.ds(cnt + L, L)] = negs
        c_s[pl.ds(cnt + 2 * L, L)] = negs
        c_s[pl.ds(cnt + 3 * L, L)] = negs
        nb2 = (cnt + (4 * L - 1)) // (4 * L)

        if t + 1 < CPW:
            @pl.when(cls + NW < NCLS)
            def _():
                for cp in descs_in[t + 1]:
                    cp.start()

        for j in range(KPAD // L):
            o_s[pl.ds(j * L, L)] = zeros

        bs0 = jnp.max(vmax)
        bi0 = jnp.min(jnp.where(vmax == bs0, vidx, 2**31 - 1))

        def cond(st):
            k, bi, bs = st
            return (k < K) & (bs > HALF_BIG_NEG)

        def body(st):
            k, bi, bs = st
            biv = jnp.full((L,), bi, jnp.int32)
            x1cv = plsc.load_gather(c_x1, [biv])
            y1cv = plsc.load_gather(c_y1, [biv])
            x2cv = plsc.load_gather(c_x2, [biv])
            y2cv = plsc.load_gather(c_y2, [biv])
            acv = plsc.load_gather(c_area, [biv])
            put1(o_s, k, bs)
            put1(o_ki, k, bi.astype(jnp.float32))
            put1(c_s, bi, BIG_NEG)

            def sup_half(off, vmax, vidx):
                sv = c_s[pl.ds(off, L)]
                x1v = c_x1[pl.ds(off, L)]
                y1v = c_y1[pl.ds(off, L)]
                x2v = c_x2[pl.ds(off, L)]
                y2v = c_y2[pl.ds(off, L)]
                av = c_area[pl.ds(off, L)]
                iw = jnp.maximum(jnp.minimum(x2cv, x2v) - jnp.maximum(x1cv, x1v), 0.0)
                ih = jnp.maximum(jnp.minimum(y2cv, y2v) - jnp.maximum(y1cv, y1v), 0.0)
                inter = iw * ih
                sup = inter > NMS_THRESH * (acv + av - inter + 1e-9)
                sv = jnp.where(sup, BIG_NEG, sv)
                c_s[pl.ds(off, L)] = sv
                upd = sv > vmax
                return jnp.where(upd, sv, vmax), jnp.where(upd, off + lanes, vidx)

            def sup_blk(b, carry):
                vmax, vidx = carry
                off = b * (4 * L)
                vmax, vidx = sup_half(off, vmax, vidx)
                vmax, vidx = sup_half(off + L, vmax, vidx)
                vmax, vidx = sup_half(off + 2 * L, vmax, vidx)
                return sup_half(off + 3 * L, vmax, vidx)

            vmax, vidx = lax.fori_loop(0, nb2, sup_blk, (ninf, bigi))
            bs2 = jnp.max(vmax)
            bi2 = jnp.min(jnp.where(vmax == bs2, vidx, 2**31 - 1))
            return k + 1, bi2, bs2

        kfin, _, _ = lax.while_loop(cond, body, (jnp.int32(0), bi0, bs0))

        for cp in descs_fut[t]:
            cp.wait()

        for j in range(KPAD // L):
            kv = j * L + lanes
            vm = kv < kfin
            kiv = o_ki[pl.ds(j * L, L)]
            biv = jnp.where(vm, kiv.astype(jnp.int32), 0)
            gof = plsc.load_gather(c_oidxf, [biv], mask=vm)
            oiv = jnp.where(vm, gof.astype(jnp.int32), 0)
            kv4 = kv * 4
            for comp, src_ref in enumerate((c_x1, c_y1, c_x2, c_y2)):
                g = plsc.load_gather(src_ref, [biv], mask=vm)
                plsc.store_scatter(o_box, [kv4 + comp], jnp.where(vm, g, 0.0))
            for comp, src_ref in enumerate((stg_f1, stg_f2, stg_f3, stg_f4)):
                g = plsc.load_gather(src_ref, [oiv], mask=vm)
                plsc.store_scatter(o_fut, [kv4 + comp], jnp.where(vm, g, 0.0))

        ocps = [
            pltpu.async_copy(o_s, os_hbm.at[cls], sem),
            pltpu.async_copy(o_box, obox_hbm.at[cls], sem),
            pltpu.async_copy(o_fut, ofut_hbm.at[cls], sem),
        ]
        if t + 1 < CPW:
            @pl.when(cls + NW < NCLS)
            def _():
                for cp in descs_fut[t + 1]:
                    cp.start()

        for cp in ocps:
            cp.wait()

    for cp in descs_in[0]:
        cp.start()
    for cp in descs_fut[0]:
        cp.start()
    for t in range(CPW):
        cls = wid + t * NW

        @pl.when(cls < NCLS)
        def _(t=t, cls=cls):
            do_class(cls, t)


def _nms_call(planes):
    mesh = plsc.VectorSubcoreMesh(core_axis_name="c", subcore_axis_name="s", num_cores=2, num_subcores=16)
    f32 = jnp.float32
    out_type = [jax.ShapeDtypeStruct((NCLS, KPAD), f32),
                jax.ShapeDtypeStruct((NCLS, KPAD * 4), f32),
                jax.ShapeDtypeStruct((NCLS, KPAD * 4), f32)]
    scratch = (
        [pltpu.VMEM((NPAD,), f32)] * 9
        + [pltpu.VMEM((NPAD + 4 * L,), f32)] * 7
        + [pltpu.VMEM((KPAD,), f32)] * 2
        + [pltpu.VMEM((KPAD * 4,), f32)] * 2
        + [pltpu.SemaphoreType.DMA] * 3
    )
    fn = pl.kernel(
        _nms_body,
        out_type=out_type,
        mesh=mesh,
        scratch_types=scratch,
        compiler_params=pltpu.CompilerParams(needs_layout_passes=False),
    )
    return fn(*planes)


def kernel(class_logit, box_regression, trajectory_regression, proposal):
    scoresT = _softmax_call(class_logit)

    pad = NPAD - N

    d3 = jnp.stack([box_regression.reshape(N, C, 4),
                    trajectory_regression.reshape(N, C, 4)])[:, :, 1:, :]
    d3 = jnp.pad(jnp.transpose(d3, (0, 3, 2, 1)),
                 ((0, 0), (0, 0), (0, 0), (0, pad)))
    deltas = (d3[0, 0], d3[0, 1], d3[0, 2], d3[0, 3])
    tdeltas = (d3[1, 0], d3[1, 1], d3[1, 2], d3[1, 3])
    propT = jnp.pad(proposal.T, ((0, 0), (0, pad)))

    planes = _prep_call(scoresT, deltas, tdeltas, propT)
    os_, boxf, futf = _nms_call(planes)

    boxes = boxf[:, :K * 4].reshape(NCLS, K, 4)
    fut = futf[:, :K * 4].reshape(NCLS, K, 4)
    scores = os_[:, :K]
    labels = jnp.broadcast_to(
        jnp.arange(1, C, dtype=jnp.int32)[:, None], (NCLS, K))
    return boxes, scores, labels, fut
